# Initial kernel scaffold; baseline (speedup 1.0000x reference)
import jax
import jax.numpy as jnp
from jax.experimental import pallas as pl


def kernel(hierarchy_labels, weight):
    raise NotImplementedError("write your pallas kernel here")


# speedup vs baseline: 3.2233x; 3.2233x over previous
import functools

import jax
import jax.numpy as jnp
from jax import lax
from jax.experimental import pallas as pl
from jax.experimental.pallas import tpu as pltpu
from jax.experimental.pallas import tpu_sc as plsc

NUM_ROWS = 17
D = 128
LANES = 16
CHUNK = 512


@functools.lru_cache(maxsize=None)
def _build(batch: int):
    info = plsc.get_sparse_core_info()
    nw = info.num_cores * info.num_subcores
    assert batch % (nw * CHUNK) == 0
    b_per_w = batch // nw
    n_chunks = b_per_w // CHUNK
    mesh = plsc.VectorSubcoreMesh(core_axis_name="c", subcore_axis_name="s")

    @functools.partial(
        pl.kernel,
        out_type=jax.ShapeDtypeStruct((batch, D), jnp.float32),
        mesh=mesh,
        scratch_types=[
            pltpu.VMEM((NUM_ROWS, D), jnp.float32),
            pltpu.VMEM((CHUNK,), jnp.int32),
            pltpu.VMEM((CHUNK, D), jnp.float32),
        ],
    )
    def gather_kernel(idx_hbm, table_hbm, out_hbm, table_v, idx_v, rows_v):
        wid = lax.axis_index("s") * info.num_cores + lax.axis_index("c")
        base = wid * b_per_w
        pltpu.sync_copy(table_hbm, table_v)

        def chunk_body(g, carry):
            off = base + g * CHUNK
            pltpu.sync_copy(idx_hbm.at[pl.ds(off, CHUNK)], idx_v)

            def group_body(t, carry2):
                vidx = idx_v[pl.ds(t * LANES, LANES)]
                for i in range(LANES):
                    r = vidx[i]
                    row = t * LANES + i
                    for j in range(D // LANES):
                        sl = pl.ds(j * LANES, LANES)
                        rows_v[row, sl] = table_v[r, sl]
                return carry2

            lax.fori_loop(0, CHUNK // LANES, group_body, 0)
            pltpu.sync_copy(rows_v, out_hbm.at[pl.ds(off, CHUNK)])
            return carry

        lax.fori_loop(0, n_chunks, chunk_body, 0)

    return gather_kernel


def kernel(hierarchy_labels, weight):
    b, l = hierarchy_labels.shape
    idx = hierarchy_labels.reshape(-1).astype(jnp.int32)
    out = _build(b * l)(idx, weight)
    return out.reshape(b, l, D)
